# Initial kernel scaffold; baseline (speedup 1.0000x reference)
import functools

import jax
import jax.numpy as jnp
from jax import lax
from jax.experimental import pallas as pl
from jax.experimental.pallas import tpu as pltpu
from jax.experimental.pallas import tpu_sc as plsc

_NC = 2
_NS = 16
_NW = _NC * _NS
_GROUP = 128


@functools.partial(jax.jit, static_argnames=("n_total", "dim"))
def _gather(idx_flat, weight, n_total, dim):
    n_per_w = n_total // _NW
    n_groups = n_per_w // _GROUP
    idx_3d = idx_flat.reshape(_NW, n_groups, _GROUP)

    mesh = plsc.VectorSubcoreMesh(core_axis_name="c", subcore_axis_name="s")

    @functools.partial(
        pl.kernel,
        mesh=mesh,
        out_type=jax.ShapeDtypeStruct((n_total, dim), jnp.float32),
        scratch_types=[
            pltpu.VMEM((n_groups, _GROUP), jnp.int32),
            pltpu.VMEM((_GROUP, dim), jnp.float32),
            pltpu.SemaphoreType.DMA,
        ],
    )
    def k(idx_hbm, table_hbm, out_hbm, idx_v, rows_v, sem):
        wid = lax.axis_index("s") * _NC + lax.axis_index("c")
        base = wid * n_per_w
        pltpu.sync_copy(idx_hbm.at[wid], idx_v)

        def body(j, _):
            pltpu.async_copy(table_hbm.at[idx_v.at[j]], rows_v, sem).wait()
            pltpu.sync_copy(rows_v, out_hbm.at[pl.ds(base + j * _GROUP, _GROUP)])
            return 0

        lax.fori_loop(0, n_groups, body, 0)

    return k(idx_3d, weight)


def kernel(input_, weight):
    b, s = input_.shape
    dim = weight.shape[1]
    n_total = b * s
    out = _gather(input_.reshape(n_total), weight, n_total, dim)
    return out.reshape(b, s, dim)


# speedup vs baseline: 1.6852x; 1.6852x over previous
import functools

import jax
import jax.numpy as jnp
from jax import lax
from jax.experimental import pallas as pl
from jax.experimental.pallas import tpu as pltpu
from jax.experimental.pallas import tpu_sc as plsc

_NC = 2
_NS = 16
_NW = _NC * _NS
_GROUP = 128


@functools.partial(jax.jit, static_argnames=("n_total", "dim"))
def _gather(idx_flat, weight, n_total, dim):
    n_per_w = n_total // _NW
    n_groups = n_per_w // _GROUP
    idx_3d = idx_flat.reshape(_NW, n_groups, _GROUP)

    mesh = plsc.VectorSubcoreMesh(core_axis_name="c", subcore_axis_name="s")

    @functools.partial(
        pl.kernel,
        mesh=mesh,
        out_type=jax.ShapeDtypeStruct((n_total, dim), jnp.float32),
        scratch_types=[
            pltpu.VMEM((n_groups, _GROUP), jnp.int32),
            pltpu.VMEM((_GROUP, dim), jnp.float32),
            pltpu.SemaphoreType.DMA,
        ],
        compiler_params=pltpu.CompilerParams(use_tc_tiling_on_sc=False),
    )
    def k(idx_hbm, table_hbm, out_hbm, idx_v, rows_v, sem):
        wid = lax.axis_index("s") * _NC + lax.axis_index("c")
        base = wid * n_per_w
        pltpu.sync_copy(idx_hbm.at[wid], idx_v)

        def body(j, _):
            pltpu.async_copy(table_hbm.at[idx_v.at[j]], rows_v, sem).wait()
            pltpu.sync_copy(rows_v, out_hbm.at[pl.ds(base + j * _GROUP, _GROUP)])
            return 0

        lax.fori_loop(0, n_groups, body, 0)

    return k(idx_3d, weight)


def kernel(input_, weight):
    b, s = input_.shape
    dim = weight.shape[1]
    n_total = b * s
    out = _gather(input_.reshape(n_total), weight, n_total, dim)
    return out.reshape(b, s, dim)


# speedup vs baseline: 1.8734x; 1.1116x over previous
import functools

import jax
import jax.numpy as jnp
from jax import lax
from jax.experimental import pallas as pl
from jax.experimental.pallas import tpu as pltpu
from jax.experimental.pallas import tpu_sc as plsc

_NC = 2
_NS = 16
_NW = _NC * _NS
_GROUP = 128
_K = 4
_BLOCK = _GROUP * _K


@functools.partial(jax.jit, static_argnames=("n_total", "dim"))
def _gather(idx_flat, weight, n_total, dim):
    n_per_w = n_total // _NW
    n_groups = n_per_w // _GROUP
    nblk = n_groups // _K
    assert nblk % 2 == 0
    idx_3d = idx_flat.reshape(_NW, n_groups, _GROUP)

    mesh = plsc.VectorSubcoreMesh(core_axis_name="c", subcore_axis_name="s")

    @functools.partial(
        pl.kernel,
        mesh=mesh,
        out_type=jax.ShapeDtypeStruct((n_total, dim), jnp.float32),
        scratch_types=[
            pltpu.VMEM((n_groups, _GROUP), jnp.int32),
            pltpu.VMEM((_BLOCK, dim), jnp.float32),
            pltpu.VMEM((_BLOCK, dim), jnp.float32),
            pltpu.SemaphoreType.DMA,
            pltpu.SemaphoreType.DMA,
            pltpu.SemaphoreType.DMA,
            pltpu.SemaphoreType.DMA,
        ],
        compiler_params=pltpu.CompilerParams(use_tc_tiling_on_sc=False),
    )
    def k(idx_hbm, table_hbm, out_hbm, idx_v, rows0, rows1, g0, g1, o0, o1):
        wid = lax.axis_index("s") * _NC + lax.axis_index("c")
        base = wid * n_per_w
        pltpu.sync_copy(idx_hbm.at[wid], idx_v)

        def gather_descs(blk, rows, gsem):
            return [
                (table_hbm.at[idx_v.at[blk * _K + b]],
                 rows.at[pl.ds(b * _GROUP, _GROUP)],
                 gsem)
                for b in range(_K)
            ]

        def out_slice(blk):
            return out_hbm.at[pl.ds(base + blk * _BLOCK, _BLOCK)]

        for src, dst, sem in gather_descs(0, rows0, g0):
            pltpu.async_copy(src, dst, sem)

        def handle(i, rows_p, gsem_p, osem_p, rows_q, gsem_q, osem_q):
            for src, dst, sem in gather_descs(i, rows_p, gsem_p):
                pltpu.make_async_copy(src, dst, sem).wait()
            pltpu.async_copy(rows_p, out_slice(i), osem_p)

            @pl.when(i + 1 < nblk)
            def _():
                @pl.when(i >= 1)
                def _():
                    pltpu.make_async_copy(rows_q, out_slice(i - 1), osem_q).wait()

                for src, dst, sem in gather_descs(i + 1, rows_q, gsem_q):
                    pltpu.async_copy(src, dst, sem)

        def body(i, _):
            even = (i % 2) == 0

            @pl.when(even)
            def _():
                handle(i, rows0, g0, o0, rows1, g1, o1)

            @pl.when(jnp.logical_not(even))
            def _():
                handle(i, rows1, g1, o1, rows0, g0, o0)

            return 0

        lax.fori_loop(0, nblk, body, 0)
        pltpu.make_async_copy(rows0, out_slice(nblk - 2), o0).wait()
        pltpu.make_async_copy(rows1, out_slice(nblk - 1), o1).wait()

    return k(idx_3d, weight)


def kernel(input_, weight):
    b, s = input_.shape
    dim = weight.shape[1]
    n_total = b * s
    out = _gather(input_.reshape(n_total), weight, n_total, dim)
    return out.reshape(b, s, dim)
